# baseline (speedup 1.0000x reference)
import functools

import jax
import jax.numpy as jnp
from jax import lax
from jax.experimental import pallas as pl
from jax.experimental.pallas import tpu as pltpu
from jax.experimental.pallas import tpu_sc as plsc

N = 10000
E = 160000
D_IN = 256
D_HID = 256
D_OUT = 128
BW = 128

EPT = 5000
CH = 96
NCH = 52
TCH = EPT - CH * NCH
DCH = 40
DNCH = 125


def _make_mesh():
    return plsc.VectorSubcoreMesh(
        core_axis_name="c", subcore_axis_name="s", num_cores=2, num_subcores=16
    )


def _over_my_rows(sid, fn):
    @pl.when(sid < 15)
    def _():
        fn(pl.multiple_of(sid * 632, 8), 632)

    @pl.when(sid == 15)
    def _():
        fn(15 * 632, N - 15 * 632)


@functools.cache
def _make_deg_kernel():
    @functools.partial(
        pl.kernel,
        out_type=jax.ShapeDtypeStruct((2, N), jnp.float32),
        mesh=_make_mesh(),
        scratch_types=[
            pltpu.VMEM((DNCH, DCH), jnp.int32),
            pltpu.VMEM((DCH,), jnp.float32),
            pltpu.VMEM_SHARED((N,), jnp.float32),
        ],
    )
    def deg_kernel(dstr, zeros_n, out, dstb, ones_u, deg_sh):
        cid = lax.axis_index("c")
        sid = lax.axis_index("s")
        pltpu.sync_copy(dstr.at[cid, sid], dstb)
        for off in (0, 16, 24):
            ones_u[pl.ds(off, 16)] = jnp.full((16,), 1.0, jnp.float32)

        @pl.when(sid == 0)
        def _():
            pltpu.sync_copy(zeros_n, deg_sh)

        plsc.subcore_barrier()

        def body(j, carry):
            pltpu.sync_copy(ones_u, deg_sh.at[dstb.at[j]], add=True)
            return carry

        lax.fori_loop(0, DNCH, body, 0)
        plsc.subcore_barrier()

        @pl.when(sid == 0)
        def _():
            pltpu.sync_copy(deg_sh, out.at[cid])

    return deg_kernel


@functools.cache
def _make_edge_sum():
    @functools.partial(
        pl.kernel,
        out_type=jax.ShapeDtypeStruct((2, N, BW), jnp.float32),
        mesh=_make_mesh(),
        scratch_types=[
            pltpu.VMEM((5008,), jnp.int32),
            pltpu.VMEM((NCH, CH), jnp.int32),
            pltpu.VMEM((1, TCH), jnp.int32),
            pltpu.VMEM((16,), jnp.int32),
            pltpu.VMEM((CH, BW), jnp.float32),
            pltpu.VMEM((CH, BW), jnp.float32),
            pltpu.VMEM((CH, BW), jnp.float32),
            pltpu.VMEM_SHARED((N, BW), jnp.float32),
            pltpu.SemaphoreType.DMA,
            pltpu.SemaphoreType.DMA,
            pltpu.SemaphoreType.DMA,
        ],
    )
    def edge_sum(hflat, zer, srcr, dstr, dstt, qb, out,
                 srcb, dstb, dstbt, qv, g0, g1, g2, s_sh,
                 sem0, sem1, sem2):
        cid = lax.axis_index("c")
        sid = lax.axis_index("s")
        pltpu.sync_copy(srcr.at[cid, sid], srcb)
        pltpu.sync_copy(dstr.at[cid, sid], dstb)
        pltpu.sync_copy(dstt.at[cid, sid], dstbt)
        pltpu.sync_copy(qb, qv)
        _over_my_rows(sid, lambda off, size: pltpu.sync_copy(
            zer.at[pl.ds(0, size)], s_sh.at[pl.ds(off, size)]))

        qvec = qv[...]

        def bias(i, carry):
            sl = pl.ds(pl.multiple_of(i * 16, 16), 16)
            srcb[sl] = srcb[sl] + qvec
            return carry

        lax.fori_loop(0, (EPT + 15) // 16, bias, 0)
        plsc.subcore_barrier()

        bufs = ((g0, sem0), (g1, sem1), (g2, sem2))
        NB = len(bufs)

        def gather(j, gb, sem):
            idx = srcb.at[pl.ds(pl.multiple_of(j * CH, 8), CH)]
            pltpu.async_copy(hflat.at[idx], gb, sem)

        def gwait(j, gb, sem):
            idx = srcb.at[pl.ds(pl.multiple_of(j * CH, 8), CH)]
            pltpu.make_async_copy(hflat.at[idx], gb, sem).wait()

        for b, (gb, sem) in enumerate(bufs):
            gather(b, gb, sem)

        def ring(i, carry):
            for b, (gb, sem) in enumerate(bufs):
                j = i * NB + b
                gwait(j, gb, sem)
                pltpu.sync_copy(gb, s_sh.at[dstb.at[j]], add=True)
                jn = j + NB

                @pl.when(jn < NCH)
                def _():
                    gather(jn, gb, sem)

            return carry

        lax.fori_loop(0, NCH // NB, ring, 0)
        tidx = srcb.at[pl.ds(NCH * CH, TCH)]
        lo = (NCH // NB) * NB
        gt = bufs[(NCH - lo) % NB][0].at[pl.ds(0, TCH)]
        tsem = bufs[(NCH - lo) % NB][1]
        pltpu.async_copy(hflat.at[tidx], gt, tsem)
        for j in range(lo, NCH):
            gb, sem = bufs[j - lo]
            gwait(j, gb, sem)
            pltpu.sync_copy(gb, s_sh.at[dstb.at[j]], add=True)
        pltpu.make_async_copy(hflat.at[tidx], gt, tsem).wait()
        pltpu.sync_copy(gt, s_sh.at[dstbt.at[0]], add=True)

        plsc.subcore_barrier()
        _over_my_rows(sid, lambda off, size: pltpu.sync_copy(
            s_sh.at[pl.ds(off, size)], out.at[cid, pl.ds(off, size)]))

    return edge_sum


ROWS = 400
GR = N // ROWS


def _mma_body(x_ref, w_ref, h_ref):
    h_ref[...] = jnp.dot(
        x_ref[...], w_ref[...], preferred_element_type=jnp.float32)


_mma = pl.pallas_call(
    _mma_body,
    grid=(GR,),
    in_specs=[
        pl.BlockSpec((ROWS, D_IN), lambda r: (r, 0)),
        pl.BlockSpec((D_IN, D_HID), lambda r: (0, 0)),
    ],
    out_specs=pl.BlockSpec((ROWS, D_HID), lambda r: (r, 0)),
    out_shape=jax.ShapeDtypeStruct((N, D_HID), jnp.float32),
)


def _mm1_body(deg_ref, h_ref, h1b_ref, dinv_ref):
    d = deg_ref[:, 0:1] + deg_ref[:, 1:2] + 1.0
    di = lax.rsqrt(d)
    h = h_ref[...] * di
    h1b_ref[0] = h[:, :BW]
    h1b_ref[1] = h[:, BW:]
    dinv_ref[...] = di


_mm1 = pl.pallas_call(
    _mm1_body,
    grid=(GR,),
    in_specs=[
        pl.BlockSpec((ROWS, 2), lambda r: (r, 0)),
        pl.BlockSpec((ROWS, D_HID), lambda r: (r, 0)),
    ],
    out_specs=[
        pl.BlockSpec((2, ROWS, BW), lambda r: (0, r, 0)),
        pl.BlockSpec((ROWS, 1), lambda r: (r, 0)),
    ],
    out_shape=[
        jax.ShapeDtypeStruct((2, N, BW), jnp.float32),
        jax.ShapeDtypeStruct((N, 1), jnp.float32),
    ],
)


def _mm2a_body(p0_ref, h1_ref, dinv_ref, w_ref, acc_ref):
    di = dinv_ref[...]
    z0 = jnp.maximum((p0_ref[0] + p0_ref[1] + h1_ref[0]) * di, 0.0)
    acc_ref[...] = jnp.dot(z0, w_ref[...], preferred_element_type=jnp.float32)


_mm2a = pl.pallas_call(
    _mm2a_body,
    grid=(GR,),
    in_specs=[
        pl.BlockSpec((2, ROWS, BW), lambda r: (0, r, 0)),
        pl.BlockSpec((2, ROWS, BW), lambda r: (0, r, 0)),
        pl.BlockSpec((ROWS, 1), lambda r: (r, 0)),
        pl.BlockSpec((BW, D_OUT), lambda r: (0, 0)),
    ],
    out_specs=pl.BlockSpec((ROWS, D_OUT), lambda r: (r, 0)),
    out_shape=jax.ShapeDtypeStruct((N, D_OUT), jnp.float32),
)


def _mm2b_body(p1_ref, h1_ref, dinv_ref, w_ref, acc_ref, o_ref):
    di = dinv_ref[...]
    z1 = jnp.maximum((p1_ref[0] + p1_ref[1] + h1_ref[1]) * di, 0.0)
    h2 = (acc_ref[...] + jnp.dot(
        z1, w_ref[...], preferred_element_type=jnp.float32)) * di
    o_ref[0] = h2
    o_ref[1] = h2


_mm2b = pl.pallas_call(
    _mm2b_body,
    grid=(GR,),
    in_specs=[
        pl.BlockSpec((2, ROWS, BW), lambda r: (0, r, 0)),
        pl.BlockSpec((2, ROWS, BW), lambda r: (0, r, 0)),
        pl.BlockSpec((ROWS, 1), lambda r: (r, 0)),
        pl.BlockSpec((BW, D_OUT), lambda r: (0, 0)),
        pl.BlockSpec((ROWS, D_OUT), lambda r: (r, 0)),
    ],
    out_specs=pl.BlockSpec((2, ROWS, BW), lambda r: (0, r, 0)),
    out_shape=jax.ShapeDtypeStruct((2, N, BW), jnp.float32),
)


def _fin_body(p2_ref, h2_ref, dinv_ref, o_ref):
    di = dinv_ref[...]
    o_ref[...] = (p2_ref[0] + p2_ref[1] + h2_ref[0]) * di


_fin = pl.pallas_call(
    _fin_body,
    grid=(GR,),
    in_specs=[
        pl.BlockSpec((2, ROWS, BW), lambda r: (0, r, 0)),
        pl.BlockSpec((1, ROWS, BW), lambda r: (0, r, 0)),
        pl.BlockSpec((ROWS, 1), lambda r: (r, 0)),
    ],
    out_specs=pl.BlockSpec((ROWS, D_OUT), lambda r: (r, 0)),
    out_shape=jax.ShapeDtypeStruct((N, D_OUT), jnp.float32),
)


def kernel(x, edge_index, W1, W2):
    src = edge_index[0].astype(jnp.int32)
    dst = edge_index[1].astype(jnp.int32)
    srcr = jnp.pad(src.reshape(32, EPT), ((0, 0), (0, 8))).reshape(2, 16, 5008)
    dst3 = dst.reshape(2, 16, EPT)
    dstr_deg = dst3.reshape(2, 16, DNCH, DCH)
    dstr = dst3[:, :, :NCH * CH].reshape(2, 16, NCH, CH)
    dstt = dst3[:, :, NCH * CH:].reshape(2, 16, 1, TCH)
    zeros_n = jnp.zeros((N,), jnp.float32)
    zer = jnp.zeros((632, BW), jnp.float32)
    qb0 = jnp.zeros((16,), jnp.int32)
    qb1 = jnp.full((16,), N, jnp.int32)

    degp = _make_deg_kernel()(dstr_deg, zeros_n)
    h1 = _mma(x, W1)
    h1b, dinv = _mm1(degp.T, h1)
    es = _make_edge_sum()
    hflat1 = h1b.reshape(2 * N, BW)
    p0 = es(hflat1, zer, srcr, dstr, dstt, qb0)
    acc = _mm2a(p0, h1b, dinv, W2[:BW])
    p1 = es(hflat1, zer, srcr, dstr, dstt, qb1)
    h2b = _mm2b(p1, h1b, dinv, W2[BW:], acc)
    p2 = es(h2b.reshape(2 * N, BW), zer, srcr, dstr, dstt, qb0)
    return _fin(p2, h2b, dinv)


# speedup vs baseline: 1.0582x; 1.0582x over previous
import functools

import jax
import jax.numpy as jnp
from jax import lax
from jax.experimental import pallas as pl
from jax.experimental.pallas import tpu as pltpu
from jax.experimental.pallas import tpu_sc as plsc

N = 10000
E = 160000
D_IN = 256
D_HID = 256
D_OUT = 128
BW = 128

EPT = 5000
CH = 96
NCH = 52
TCH = EPT - CH * NCH


def _make_mesh():
    return plsc.VectorSubcoreMesh(
        core_axis_name="c", subcore_axis_name="s", num_cores=2, num_subcores=16
    )


def _over_my_rows(sid, fn):
    @pl.when(sid < 15)
    def _():
        fn(pl.multiple_of(sid * 632, 8), 632)

    @pl.when(sid == 15)
    def _():
        fn(15 * 632, N - 15 * 632)


@functools.cache
def _make_deg_kernel():
    @functools.partial(
        pl.kernel,
        out_type=jax.ShapeDtypeStruct((2, N), jnp.float32),
        mesh=_make_mesh(),
        scratch_types=[
            pltpu.VMEM((NCH, CH), jnp.int32),
            pltpu.VMEM((1, TCH), jnp.int32),
            pltpu.VMEM((CH,), jnp.float32),
            pltpu.VMEM_SHARED((N,), jnp.float32),
        ],
    )
    def deg_kernel(dstr, dstt, zeros_n, out, dstb, dstbt, ones_u, deg_sh):
        cid = lax.axis_index("c")
        sid = lax.axis_index("s")
        pltpu.sync_copy(dstr.at[cid, sid], dstb)
        pltpu.sync_copy(dstt.at[cid, sid], dstbt)
        for off in range(0, CH, 16):
            ones_u[pl.ds(off, 16)] = jnp.full((16,), 1.0, jnp.float32)

        @pl.when(sid == 0)
        def _():
            pltpu.sync_copy(zeros_n, deg_sh)

        plsc.subcore_barrier()

        def body(j, carry):
            pltpu.sync_copy(ones_u, deg_sh.at[dstb.at[j]], add=True)
            return carry

        lax.fori_loop(0, NCH, body, 0)
        pltpu.sync_copy(ones_u.at[pl.ds(0, TCH)], deg_sh.at[dstbt.at[0]],
                        add=True)
        plsc.subcore_barrier()

        @pl.when(sid == 0)
        def _():
            pltpu.sync_copy(deg_sh, out.at[cid])

    return deg_kernel


@functools.cache
def _make_edge_sum():
    @functools.partial(
        pl.kernel,
        out_type=jax.ShapeDtypeStruct((2, N, BW), jnp.float32),
        mesh=_make_mesh(),
        scratch_types=[
            pltpu.VMEM((5008,), jnp.int32),
            pltpu.VMEM((NCH, CH), jnp.int32),
            pltpu.VMEM((1, TCH), jnp.int32),
            pltpu.VMEM((16,), jnp.int32),
            pltpu.VMEM((CH, BW), jnp.float32),
            pltpu.VMEM((CH, BW), jnp.float32),
            pltpu.VMEM((CH, BW), jnp.float32),
            pltpu.VMEM_SHARED((N, BW), jnp.float32),
            pltpu.SemaphoreType.DMA,
            pltpu.SemaphoreType.DMA,
            pltpu.SemaphoreType.DMA,
        ],
    )
    def edge_sum(hflat, zer, srcr, dstr, dstt, qb, out,
                 srcb, dstb, dstbt, qv, g0, g1, g2, s_sh,
                 sem0, sem1, sem2):
        cid = lax.axis_index("c")
        sid = lax.axis_index("s")
        pltpu.sync_copy(srcr.at[cid, sid], srcb)
        pltpu.sync_copy(dstr.at[cid, sid], dstb)
        pltpu.sync_copy(dstt.at[cid, sid], dstbt)
        pltpu.sync_copy(qb, qv)
        _over_my_rows(sid, lambda off, size: pltpu.sync_copy(
            zer.at[pl.ds(0, size)], s_sh.at[pl.ds(off, size)]))

        qvec = qv[...]

        def bias(i, carry):
            sl = pl.ds(pl.multiple_of(i * 16, 16), 16)
            srcb[sl] = srcb[sl] + qvec
            return carry

        lax.fori_loop(0, (EPT + 15) // 16, bias, 0)
        plsc.subcore_barrier()

        bufs = ((g0, sem0), (g1, sem1), (g2, sem2))
        NB = len(bufs)

        def gather(j, gb, sem):
            idx = srcb.at[pl.ds(pl.multiple_of(j * CH, 8), CH)]
            pltpu.async_copy(hflat.at[idx], gb, sem)

        def gwait(j, gb, sem):
            idx = srcb.at[pl.ds(pl.multiple_of(j * CH, 8), CH)]
            pltpu.make_async_copy(hflat.at[idx], gb, sem).wait()

        for b, (gb, sem) in enumerate(bufs):
            gather(b, gb, sem)

        def ring(i, carry):
            for b, (gb, sem) in enumerate(bufs):
                j = i * NB + b
                gwait(j, gb, sem)
                pltpu.sync_copy(gb, s_sh.at[dstb.at[j]], add=True)
                jn = j + NB

                @pl.when(jn < NCH)
                def _():
                    gather(jn, gb, sem)

            return carry

        lax.fori_loop(0, NCH // NB, ring, 0)
        tidx = srcb.at[pl.ds(NCH * CH, TCH)]
        lo = (NCH // NB) * NB
        gt = bufs[(NCH - lo) % NB][0].at[pl.ds(0, TCH)]
        tsem = bufs[(NCH - lo) % NB][1]
        pltpu.async_copy(hflat.at[tidx], gt, tsem)
        for j in range(lo, NCH):
            gb, sem = bufs[j - lo]
            gwait(j, gb, sem)
            pltpu.sync_copy(gb, s_sh.at[dstb.at[j]], add=True)
        pltpu.make_async_copy(hflat.at[tidx], gt, tsem).wait()
        pltpu.sync_copy(gt, s_sh.at[dstbt.at[0]], add=True)

        plsc.subcore_barrier()
        _over_my_rows(sid, lambda off, size: pltpu.sync_copy(
            s_sh.at[pl.ds(off, size)], out.at[cid, pl.ds(off, size)]))

    return edge_sum


ROWS = 400
GR = N // ROWS


def _mm1_body(deg_ref, x_ref, w_ref, h_ref, dinv_ref):
    d = deg_ref[:, 0:1] + deg_ref[:, 1:2] + 1.0
    di = lax.rsqrt(d)
    h = jnp.dot(x_ref[...], w_ref[...], preferred_element_type=jnp.float32)
    h = h * di
    h_ref[0] = h[:, :BW]
    h_ref[1] = h[:, BW:]
    dinv_ref[...] = di


_mm1 = pl.pallas_call(
    _mm1_body,
    grid=(GR,),
    in_specs=[
        pl.BlockSpec((ROWS, 2), lambda r: (r, 0)),
        pl.BlockSpec((ROWS, D_IN), lambda r: (r, 0)),
        pl.BlockSpec((D_IN, D_HID), lambda r: (0, 0)),
    ],
    out_specs=[
        pl.BlockSpec((2, ROWS, BW), lambda r: (0, r, 0)),
        pl.BlockSpec((ROWS, 1), lambda r: (r, 0)),
    ],
    out_shape=[
        jax.ShapeDtypeStruct((2, N, BW), jnp.float32),
        jax.ShapeDtypeStruct((N, 1), jnp.float32),
    ],
)


def _mm2_body(p0_ref, p1_ref, h1_ref, dinv_ref, w_ref, o_ref):
    di = dinv_ref[...]
    z0 = (p0_ref[0] + p0_ref[1] + h1_ref[0]) * di
    z1 = (p1_ref[0] + p1_ref[1] + h1_ref[1]) * di
    z = jnp.maximum(jnp.concatenate([z0, z1], axis=1), 0.0)
    h2 = jnp.dot(z, w_ref[...], preferred_element_type=jnp.float32) * di
    o_ref[0] = h2
    o_ref[1] = h2


_mm2 = pl.pallas_call(
    _mm2_body,
    grid=(GR,),
    in_specs=[
        pl.BlockSpec((2, ROWS, BW), lambda r: (0, r, 0)),
        pl.BlockSpec((2, ROWS, BW), lambda r: (0, r, 0)),
        pl.BlockSpec((2, ROWS, BW), lambda r: (0, r, 0)),
        pl.BlockSpec((ROWS, 1), lambda r: (r, 0)),
        pl.BlockSpec((D_HID, D_OUT), lambda r: (0, 0)),
    ],
    out_specs=pl.BlockSpec((2, ROWS, BW), lambda r: (0, r, 0)),
    out_shape=jax.ShapeDtypeStruct((2, N, BW), jnp.float32),
)


def _fin_body(p2_ref, h2_ref, dinv_ref, o_ref):
    di = dinv_ref[...]
    o_ref[...] = (p2_ref[0] + p2_ref[1] + h2_ref[0]) * di


_fin = pl.pallas_call(
    _fin_body,
    grid=(GR,),
    in_specs=[
        pl.BlockSpec((2, ROWS, BW), lambda r: (0, r, 0)),
        pl.BlockSpec((1, ROWS, BW), lambda r: (0, r, 0)),
        pl.BlockSpec((ROWS, 1), lambda r: (r, 0)),
    ],
    out_specs=pl.BlockSpec((ROWS, D_OUT), lambda r: (r, 0)),
    out_shape=jax.ShapeDtypeStruct((N, D_OUT), jnp.float32),
)


def kernel(x, edge_index, W1, W2):
    src = edge_index[0].astype(jnp.int32)
    dst = edge_index[1].astype(jnp.int32)
    srcr = jnp.pad(src.reshape(32, EPT), ((0, 0), (0, 8))).reshape(2, 16, 5008)
    dst3 = dst.reshape(2, 16, EPT)
    dstr = dst3[:, :, :NCH * CH].reshape(2, 16, NCH, CH)
    dstt = dst3[:, :, NCH * CH:].reshape(2, 16, 1, TCH)
    zeros_n = jnp.zeros((N,), jnp.float32)
    zer = jnp.zeros((632, BW), jnp.float32)
    qb0 = jnp.zeros((16,), jnp.int32)
    qb1 = jnp.full((16,), N, jnp.int32)

    degp = _make_deg_kernel()(dstr, dstt, zeros_n)
    h1b, dinv = _mm1(degp.T, x, W1)
    es = _make_edge_sum()
    hflat1 = h1b.reshape(2 * N, BW)
    p0 = es(hflat1, zer, srcr, dstr, dstt, qb0)
    p1 = es(hflat1, zer, srcr, dstr, dstt, qb1)
    h2b = _mm2(p0, p1, h1b, dinv, W2)
    p2 = es(h2b.reshape(2 * N, BW), zer, srcr, dstr, dstt, qb0)
    return _fin(p2, h2b, dinv)
